# Initial kernel scaffold; baseline (speedup 1.0000x reference)
import jax
import jax.numpy as jnp
from jax.experimental import pallas as pl


def kernel(x, class_logits):
    raise NotImplementedError("write your pallas kernel here")


# speedup vs baseline: 1.0999x; 1.0999x over previous
import functools

import jax
import jax.numpy as jnp
from jax import lax
from jax.experimental import pallas as pl
from jax.experimental.pallas import tpu as pltpu

NUM_CLASSES = 100
CHANNELS = 64
H = 8
W = 8
BATCH = 1024
FEAT = CHANNELS * H * W

CLS_PAD = 104
ROW_BLK = 8


def _fused_body(x_ref, lg_ref, proto_out_ref, inter_ref, proto_scr, *, row_blk):
    i = pl.program_id(0)

    @pl.when(i == 0)
    def _init():
        lg = lg_ref[...]
        m = jnp.max(lg, axis=1, keepdims=True)
        idx2 = lax.broadcasted_iota(jnp.int32, lg.shape, 1)
        cls = jnp.min(jnp.where(lg == m, idx2, NUM_CLASSES), axis=1)
        onehot = (cls[:, None] == lax.broadcasted_iota(
            jnp.int32, (BATCH, CLS_PAD), 1)).astype(jnp.float32)
        sums = lax.dot_general(
            onehot, x_ref[...],
            dimension_numbers=(((0,), (0,)), ((), ())),
            preferred_element_type=jnp.float32)
        counts = jnp.sum(onehot, axis=0)
        denom = jnp.where(counts > 0, counts, 1.0)
        proto = sums / denom[:, None]
        proto_scr[...] = proto
        proto_out_ref[...] = proto[:NUM_CLASSES, :]

    p = proto_scr[...][:NUM_CLASSES, :]
    pi = proto_scr[pl.ds(i * row_blk, row_blk), :]
    inter_ref[...] = p[None, :, :] - pi[:, None, :]


def kernel(x, class_logits):
    xf = x.reshape(BATCH, FEAT)
    n_steps = pl.cdiv(NUM_CLASSES, ROW_BLK)
    proto, inter = pl.pallas_call(
        functools.partial(_fused_body, row_blk=ROW_BLK),
        grid=(n_steps,),
        in_specs=[
            pl.BlockSpec((BATCH, FEAT), lambda i: (0, 0)),
            pl.BlockSpec((BATCH, NUM_CLASSES), lambda i: (0, 0)),
        ],
        out_specs=[
            pl.BlockSpec((NUM_CLASSES, FEAT), lambda i: (0, 0)),
            pl.BlockSpec((ROW_BLK, NUM_CLASSES, FEAT), lambda i: (i, 0, 0)),
        ],
        out_shape=[
            jax.ShapeDtypeStruct((NUM_CLASSES, FEAT), jnp.float32),
            jax.ShapeDtypeStruct((NUM_CLASSES, NUM_CLASSES, FEAT), jnp.float32),
        ],
        scratch_shapes=[pltpu.VMEM((CLS_PAD, FEAT), jnp.float32)],
    )(xf, class_logits)
    prototypes = proto.reshape(NUM_CLASSES, CHANNELS, H, W)
    inter_class_matrix = inter.reshape(NUM_CLASSES, NUM_CLASSES, CHANNELS, H, W)
    return (prototypes, inter_class_matrix)
